# baseline (speedup 1.0000x reference)
import functools

import jax
import jax.numpy as jnp
from jax import lax
from jax.experimental import pallas as pl
from jax.experimental.pallas import tpu as pltpu
from jax.experimental.pallas import tpu_sc as plsc

B = 4
S = 2048
D = 1024
T = B * S
L = 16
NC = 2
NS = 16
NW = NC * NS
PPW = S // NW
C = 32
NCH = PPW // C
NSL = D // L
EPS = 1e-12


def _tec_body(x_h, seg_h, tok_h, segemb_h, pe_h, gamma_h, beta_h, out_h,
              idx_v, seg_v, pe_v, segemb_v, gamma_v, beta_v, tok_v, sem):
    cid = lax.axis_index("c")
    sid = lax.axis_index("s")
    w = sid * NC + cid
    pos0 = w * PPW

    pltpu.sync_copy(pe_h.at[pl.ds(pos0, PPW)], pe_v)
    pltpu.sync_copy(segemb_h, segemb_v)
    pltpu.sync_copy(gamma_h, gamma_v)
    pltpu.sync_copy(beta_h, beta_v)
    for b in range(B):
        pltpu.sync_copy(x_h.at[pl.ds(b * S + pos0, PPW)],
                        idx_v.at[pl.ds(b * PPW, PPW)])
        pltpu.sync_copy(seg_h.at[pl.ds(b * S + pos0, PPW)],
                        seg_v.at[pl.ds(b * PPW, PPW)])

    half = jnp.full((L,), 0.5, jnp.float32)
    three_half = jnp.full((L,), 1.5, jnp.float32)
    lane = lax.iota(jnp.int32, L)
    fzero = jnp.zeros((L,), jnp.float32)

    for b in range(B):
        for ci in range(NCH):
            toff = b * PPW + ci * C
            poff = ci * C
            pltpu.async_copy(tok_h.at[idx_v.at[pl.ds(toff, C)]],
                             tok_v, sem).wait()

            for gi in range(C // L):
                segf = seg_v[pl.ds(toff + gi * L, L)].astype(jnp.float32)

                def tok_fn(k, _, gi=gi, segf=segf, poff=poff):
                    i = gi * L + k
                    sg = jnp.sum(jnp.where(lane == k, segf, fzero))

                    def pass1(j, carry):
                        acc, acc2 = carry
                        sl = pl.ds(j * L, L)
                        s0 = segemb_v[0, sl]
                        e = (tok_v[i, sl] + pe_v[poff + i, sl]
                             + s0 + sg * (segemb_v[1, sl] - s0))
                        tok_v[i, sl] = e
                        return acc + e, acc2 + e * e

                    acc, acc2 = lax.fori_loop(
                        0, NSL, pass1,
                        (jnp.zeros((L,), jnp.float32),
                         jnp.zeros((L,), jnp.float32)))
                    u = jnp.sum(acc) * (1.0 / D)
                    var = jnp.sum(acc2) * (1.0 / D) - u * u
                    vv = jnp.full((L,), var + EPS, jnp.float32)
                    bits = plsc.bitcast(vv, jnp.int32)
                    y = plsc.bitcast(jnp.int32(0x5F3759DF) - (bits >> 1),
                                     jnp.float32)
                    for _ in range(3):
                        y = y * (three_half - half * vv * y * y)
                    uv = jnp.full((L,), u, jnp.float32)

                    def pass2(j, _):
                        sl = pl.ds(j * L, L)
                        e = (tok_v[i, sl] - uv) * y
                        tok_v[i, sl] = e * gamma_v[sl] + beta_v[sl]
                        return 0

                    lax.fori_loop(0, NSL, pass2, 0)
                    return 0

                lax.fori_loop(0, L, tok_fn, 0)
            pltpu.sync_copy(tok_v,
                            out_h.at[pl.ds(b * S + pos0 + ci * C, C)])


@jax.jit
def _run(x_flat, seg_flat, tok_embed, seg_embed, pe2d, gamma, beta):
    mesh = plsc.VectorSubcoreMesh(core_axis_name="c", subcore_axis_name="s")
    f = pl.kernel(
        _tec_body,
        out_type=jax.ShapeDtypeStruct((T, D), jnp.float32),
        mesh=mesh,
        scratch_types=[
            pltpu.VMEM((B * PPW,), jnp.int32),
            pltpu.VMEM((B * PPW,), jnp.int32),
            pltpu.VMEM((PPW, D), jnp.float32),
            pltpu.VMEM((2, D), jnp.float32),
            pltpu.VMEM((D,), jnp.float32),
            pltpu.VMEM((D,), jnp.float32),
            pltpu.VMEM((C, D), jnp.float32),
            pltpu.SemaphoreType.DMA,
        ],
        compiler_params=pltpu.CompilerParams(needs_layout_passes=False),
    )
    return f(x_flat, seg_flat, tok_embed, seg_embed, pe2d, gamma, beta)


def kernel(x, seg, tok_embed, seg_embed, pe, gamma, beta):
    out = _run(x.reshape(-1), seg.reshape(-1), tok_embed, seg_embed,
               pe.reshape(pe.shape[1], pe.shape[2]), gamma, beta)
    return out.reshape(x.shape[0], x.shape[1], D)


# speedup vs baseline: 1.1387x; 1.1387x over previous
import jax
import jax.numpy as jnp
from jax import lax
from jax.experimental import pallas as pl
from jax.experimental.pallas import tpu as pltpu
from jax.experimental.pallas import tpu_sc as plsc

B = 4
S = 2048
D = 1024
T = B * S
L = 16
NC = 2
NS = 16
NW = NC * NS
PPW = S // NW
C = 16
NCH = PPW // C
CHUNKS = B * NCH
NSL = D // L
EPS = 1e-12


def _tec_body(x_h, seg_h, tok_h, segemb_h, pe_h, gamma_h, beta_h, out_h,
              idx_v, seg_v, pe_v, segemb_v, gamma_v, beta_v, tok_v,
              gs0, gs1, ws0, ws1):
    cid = lax.axis_index("c")
    sid = lax.axis_index("s")
    w = sid * NC + cid
    pos0 = w * PPW
    gsem = (gs0, gs1)
    wsem = (ws0, ws1)

    pltpu.sync_copy(pe_h.at[pl.ds(pos0, PPW)], pe_v)
    pltpu.sync_copy(segemb_h, segemb_v)
    pltpu.sync_copy(gamma_h, gamma_v)
    pltpu.sync_copy(beta_h, beta_v)
    for b in range(B):
        pltpu.sync_copy(x_h.at[pl.ds(b * S + pos0, PPW)],
                        idx_v.at[pl.ds(b * PPW, PPW)])
        pltpu.sync_copy(seg_h.at[pl.ds(b * S + pos0, PPW)],
                        seg_v.at[pl.ds(b * PPW, PPW)])

    half = jnp.full((L,), 0.5, jnp.float32)
    three_half = jnp.full((L,), 1.5, jnp.float32)
    lane = lax.iota(jnp.int32, L)
    izero = jnp.zeros((L,), jnp.int32)
    fzero = jnp.zeros((L,), jnp.float32)

    def chunk_off(k):
        b, ci = divmod(k, NCH)
        return b * PPW + ci * C, ci * C, b * S + pos0 + ci * C

    def issue_gather(k, buf):
        toff, _, _ = chunk_off(k)
        return pltpu.async_copy(tok_h.at[idx_v.at[pl.ds(toff, C)]],
                                tok_v.at[buf], gsem[buf])

    def compute(k, buf):
        toff, poff, _ = chunk_off(k)
        tb = tok_v.at[buf]
        segv = seg_v[pl.ds(toff, L)]

        def tok_fn(i, _):
            sgi = jnp.sum(jnp.where(lane == i, segv, izero))

            def pass1(j, carry):
                acc, acc2 = carry
                sl = pl.ds(j * L, L)
                e = tb[i, sl] + pe_v[poff + i, sl] + segemb_v[sgi, sl]
                tb[i, sl] = e
                return acc + e, acc2 + e * e

            acc, acc2 = lax.fori_loop(0, NSL, pass1, (fzero, fzero),
                                      unroll=4)
            u = jnp.sum(acc) * (1.0 / D)
            var = jnp.sum(acc2) * (1.0 / D) - u * u
            vv = jnp.full((L,), var + EPS, jnp.float32)
            bits = plsc.bitcast(vv, jnp.int32)
            y = plsc.bitcast(jnp.int32(0x5F3759DF) - (bits >> 1),
                             jnp.float32)
            for _ in range(3):
                y = y * (three_half - half * vv * y * y)
            uv = jnp.full((L,), u, jnp.float32)

            def pass2(j, _):
                sl = pl.ds(j * L, L)
                e = (tb[i, sl] - uv) * y
                tb[i, sl] = e * gamma_v[sl] + beta_v[sl]
                return 0

            lax.fori_loop(0, NSL, pass2, 0, unroll=4)
            return 0

        lax.fori_loop(0, C, tok_fn, 0)

    write_h = [None, None]
    gather_h = [None, None]
    gather_h[0] = issue_gather(0, 0)
    for k in range(CHUNKS):
        buf = k % 2
        nbuf = 1 - buf
        if k + 1 < CHUNKS:
            if write_h[nbuf] is not None:
                write_h[nbuf].wait()
            gather_h[nbuf] = issue_gather(k + 1, nbuf)
        gather_h[buf].wait()
        compute(k, buf)
        _, _, ooff = chunk_off(k)
        write_h[buf] = pltpu.async_copy(tok_v.at[buf],
                                        out_h.at[pl.ds(ooff, C)],
                                        wsem[buf])
    write_h[0].wait()
    write_h[1].wait()


@jax.jit
def _run(x_flat, seg_flat, tok_embed, seg_embed, pe2d, gamma, beta):
    mesh = plsc.VectorSubcoreMesh(core_axis_name="c", subcore_axis_name="s")
    f = pl.kernel(
        _tec_body,
        out_type=jax.ShapeDtypeStruct((T, D), jnp.float32),
        mesh=mesh,
        scratch_types=[
            pltpu.VMEM((B * PPW,), jnp.int32),
            pltpu.VMEM((B * PPW,), jnp.int32),
            pltpu.VMEM((PPW, D), jnp.float32),
            pltpu.VMEM((2, D), jnp.float32),
            pltpu.VMEM((D,), jnp.float32),
            pltpu.VMEM((D,), jnp.float32),
            pltpu.VMEM((2, C, D), jnp.float32),
            pltpu.SemaphoreType.DMA,
            pltpu.SemaphoreType.DMA,
            pltpu.SemaphoreType.DMA,
            pltpu.SemaphoreType.DMA,
        ],
        compiler_params=pltpu.CompilerParams(needs_layout_passes=False),
    )
    return f(x_flat, seg_flat, tok_embed, seg_embed, pe2d, gamma, beta)


def kernel(x, seg, tok_embed, seg_embed, pe, gamma, beta):
    out = _run(x.reshape(-1), seg.reshape(-1), tok_embed, seg_embed,
               pe.reshape(pe.shape[1], pe.shape[2]), gamma, beta)
    return out.reshape(x.shape[0], x.shape[1], D)


# speedup vs baseline: 3.8791x; 3.4065x over previous
import jax
import jax.numpy as jnp
from jax import lax
from jax.experimental import pallas as pl
from jax.experimental.pallas import tpu as pltpu
from jax.experimental.pallas import tpu_sc as plsc

B = 4
S = 2048
D = 1024
T = B * S
NC = 2
NS = 16
NW = NC * NS
TPW = T // NW
C = 16
NBUF = 4
NCH = TPW // C
EPS = 1e-12

BP = 256
NPB = S // BP


def _sc_gather_body(x_h, tok_h, out_h, idx_v, row_v, *sems):
    cid = lax.axis_index("c")
    sid = lax.axis_index("s")
    w = sid * NC + cid
    base = w * TPW
    gsem = sems[:NBUF]
    wsem = sems[NBUF:]

    pltpu.sync_copy(x_h.at[pl.ds(base, TPW)], idx_v)

    wh = [None] * NBUF
    gh = [None] * NBUF
    for k in range(min(NBUF, NCH)):
        gh[k] = pltpu.async_copy(
            tok_h.at[idx_v.at[pl.ds(k * C, C)]], row_v.at[k], gsem[k])
    for k in range(NCH):
        b = k % NBUF
        gh[b].wait()
        wh[b] = pltpu.async_copy(
            row_v.at[b], out_h.at[pl.ds(base + k * C, C)], wsem[b])
        nk = k + NBUF
        if nk < NCH:
            wh[b].wait()
            gh[b] = pltpu.async_copy(
                tok_h.at[idx_v.at[pl.ds(nk * C, C)]], row_v.at[b], gsem[b])
    for b in range(min(NBUF, NCH)):
        if wh[b] is not None:
            wh[b].wait()


@jax.jit
def _run(x_flat, seg2d, tok_embed, seg_embed, pe2d, gamma2, beta2):
    mesh = plsc.VectorSubcoreMesh(core_axis_name="c", subcore_axis_name="s")
    gathered = pl.kernel(
        _sc_gather_body,
        out_type=jax.ShapeDtypeStruct((T, D), jnp.float32),
        mesh=mesh,
        scratch_types=[
            pltpu.VMEM((TPW,), jnp.int32),
            pltpu.VMEM((NBUF, C, D), jnp.float32),
        ] + [pltpu.SemaphoreType.DMA] * (2 * NBUF),
        compiler_params=pltpu.CompilerParams(needs_layout_passes=False),
    )(x_flat, tok_embed)

    def tc_body(gath, pe, seg, segemb, gamma, beta, out):
        sf = seg[...].astype(jnp.float32)
        s0 = segemb[0:1, :]
        s1 = segemb[1:2, :]
        e = gath[...] + pe[...] + s0 + sf * (s1 - s0)
        u = jnp.mean(e, axis=-1, keepdims=True)
        d = e - u
        var = jnp.mean(d * d, axis=-1, keepdims=True)
        out[...] = d * lax.rsqrt(var + EPS) * gamma[...] + beta[...]

    out = pl.pallas_call(
        tc_body,
        grid=(NPB, B),
        in_specs=[
            pl.BlockSpec((BP, D), lambda p, b: (b * NPB + p, 0)),
            pl.BlockSpec((BP, D), lambda p, b: (p, 0)),
            pl.BlockSpec((BP, 1), lambda p, b: (b * NPB + p, 0)),
            pl.BlockSpec((2, D), lambda p, b: (0, 0)),
            pl.BlockSpec((1, D), lambda p, b: (0, 0)),
            pl.BlockSpec((1, D), lambda p, b: (0, 0)),
        ],
        out_specs=pl.BlockSpec((BP, D), lambda p, b: (b * NPB + p, 0)),
        out_shape=jax.ShapeDtypeStruct((T, D), jnp.float32),
    )(gathered, pe2d, seg2d, seg_embed, gamma2, beta2)
    return out


def kernel(x, seg, tok_embed, seg_embed, pe, gamma, beta):
    out = _run(x.reshape(-1), seg.reshape(-1, 1), tok_embed, seg_embed,
               pe.reshape(pe.shape[1], pe.shape[2]),
               gamma.reshape(1, D), beta.reshape(1, D))
    return out.reshape(x.shape[0], x.shape[1], D)


# speedup vs baseline: 4.3399x; 1.1188x over previous
import jax
import jax.numpy as jnp
from jax import lax
from jax.experimental import pallas as pl
from jax.experimental.pallas import tpu as pltpu
from jax.experimental.pallas import tpu_sc as plsc

B = 4
S = 2048
D = 1024
T = B * S
NC = 2
NS = 16
NW = NC * NS
TPW = T // NW
C = 16
NBUF = 4
NCH = TPW // C
EPS = 1e-12

BP = 512
NPB = S // BP


def _sc_gather_body(x_h, tok_h, out_h, idx_v, row_v, *sems):
    cid = lax.axis_index("c")
    sid = lax.axis_index("s")
    w = sid * NC + cid
    base = w * TPW
    gsem = sems[:NBUF]
    wsem = sems[NBUF:]

    pltpu.sync_copy(x_h.at[pl.ds(base, TPW)], idx_v)

    wh = [None] * NBUF
    gh = [None] * NBUF
    for k in range(min(NBUF, NCH)):
        gh[k] = pltpu.async_copy(
            tok_h.at[idx_v.at[pl.ds(k * C, C)]], row_v.at[k], gsem[k])
    for k in range(NCH):
        b = k % NBUF
        gh[b].wait()
        wh[b] = pltpu.async_copy(
            row_v.at[b], out_h.at[pl.ds(base + k * C, C)], wsem[b])
        nk = k + NBUF
        if nk < NCH:
            wh[b].wait()
            gh[b] = pltpu.async_copy(
                tok_h.at[idx_v.at[pl.ds(nk * C, C)]], row_v.at[b], gsem[b])
    for b in range(min(NBUF, NCH)):
        if wh[b] is not None:
            wh[b].wait()


@jax.jit
def _run(x_flat, seg2d, tok_embed, seg_embed, pe2d, gamma2, beta2):
    mesh = plsc.VectorSubcoreMesh(core_axis_name="c", subcore_axis_name="s")
    gathered = pl.kernel(
        _sc_gather_body,
        out_type=jax.ShapeDtypeStruct((T, D), jnp.float32),
        mesh=mesh,
        scratch_types=[
            pltpu.VMEM((TPW,), jnp.int32),
            pltpu.VMEM((NBUF, C, D), jnp.float32),
        ] + [pltpu.SemaphoreType.DMA] * (2 * NBUF),
        compiler_params=pltpu.CompilerParams(needs_layout_passes=False),
    )(x_flat, tok_embed)

    def tc_body(gath, pe, seg, segemb, gamma, beta, out):
        sf = seg[...]
        s0 = segemb[0:1, :]
        s1 = segemb[1:2, :]
        e = gath[...] + pe[...] + s0 + sf * (s1 - s0)
        u = jnp.mean(e, axis=-1, keepdims=True)
        d = e - u
        var = jnp.mean(d * d, axis=-1, keepdims=True)
        out[...] = d * lax.rsqrt(var + EPS) * gamma[...] + beta[...]

    out = pl.pallas_call(
        tc_body,
        grid=(NPB, B),
        in_specs=[
            pl.BlockSpec((BP, D), lambda p, b: (b * NPB + p, 0)),
            pl.BlockSpec((BP, D), lambda p, b: (p, 0)),
            pl.BlockSpec((BP, 1), lambda p, b: (b * NPB + p, 0)),
            pl.BlockSpec((2, D), lambda p, b: (0, 0)),
            pl.BlockSpec((1, D), lambda p, b: (0, 0)),
            pl.BlockSpec((1, D), lambda p, b: (0, 0)),
        ],
        out_specs=pl.BlockSpec((BP, D), lambda p, b: (b * NPB + p, 0)),
        out_shape=jax.ShapeDtypeStruct((T, D), jnp.float32),
    )(gathered, pe2d, seg2d, seg_embed, gamma2, beta2)
    return out


def kernel(x, seg, tok_embed, seg_embed, pe, gamma, beta):
    out = _run(x.reshape(-1), seg.astype(jnp.float32).reshape(-1, 1),
               tok_embed, seg_embed,
               pe.reshape(pe.shape[1], pe.shape[2]),
               gamma.reshape(1, D), beta.reshape(1, D))
    return out.reshape(x.shape[0], x.shape[1], D)
